# Initial kernel scaffold; baseline (speedup 1.0000x reference)
import jax
import jax.numpy as jnp
from jax.experimental import pallas as pl


def kernel(input_len, pe_table):
    raise NotImplementedError("write your pallas kernel here")


# speedup vs baseline: 2.9075x; 2.9075x over previous
import jax
import jax.numpy as jnp
from jax.experimental import pallas as pl
from jax.experimental.pallas import tpu as pltpu

D_MODEL = 1024
MAX_SEQ_LEN = 2048
BATCH = 16
TJ = 256


def _emb_body(len_ref, pe_ref, emb_ref, pos_ref):
    j = pl.program_id(0)
    b = pl.program_id(1)
    len_b = len_ref[b]

    rows = j * TJ + jax.lax.broadcasted_iota(jnp.int32, (TJ, 1), 0)
    mask = rows < len_b
    emb_ref[0] = jnp.where(mask, pe_ref[...], 0.0)

    col = jax.lax.broadcasted_iota(jnp.int32, (1, 1, MAX_SEQ_LEN), 2)
    pos_ref[...] = jnp.where(col < len_b, col + 1, 0)


def kernel(input_len, pe_table):
    pe = pe_table[1:]
    nj = MAX_SEQ_LEN // TJ
    emb, pos = pl.pallas_call(
        _emb_body,
        grid=(nj, BATCH),
        in_specs=[
            pl.BlockSpec(memory_space=pltpu.SMEM),
            pl.BlockSpec((TJ, D_MODEL), lambda j, b: (j, 0)),
        ],
        out_specs=[
            pl.BlockSpec((1, TJ, D_MODEL), lambda j, b: (b, j, 0)),
            pl.BlockSpec((1, 1, MAX_SEQ_LEN), lambda j, b: (b, 0, 0)),
        ],
        out_shape=[
            jax.ShapeDtypeStruct((BATCH, MAX_SEQ_LEN, D_MODEL), jnp.float32),
            jax.ShapeDtypeStruct((BATCH, 1, MAX_SEQ_LEN), jnp.int32),
        ],
    )(input_len, pe)
    return (emb, pos.reshape(BATCH, MAX_SEQ_LEN))


# speedup vs baseline: 3.9271x; 1.3507x over previous
import jax
import jax.numpy as jnp
from jax.experimental import pallas as pl
from jax.experimental.pallas import tpu as pltpu

D_MODEL = 1024
MAX_SEQ_LEN = 2048
BATCH = 16
TJ = 512


def _emb_body(len_ref, pe_ref, emb_ref, pos_ref):
    j = pl.program_id(0)
    b = pl.program_id(1)
    len_b = len_ref[b]

    @pl.when(len_b >= (j + 1) * TJ)
    def _copy():
        emb_ref[0] = pe_ref[...]

    @pl.when(len_b <= j * TJ)
    def _zero():
        emb_ref[0] = jnp.zeros((TJ, D_MODEL), jnp.float32)

    @pl.when(jnp.logical_and(len_b > j * TJ, len_b < (j + 1) * TJ))
    def _straddle():
        rows = j * TJ + jax.lax.broadcasted_iota(jnp.int32, (TJ, 1), 0)
        emb_ref[0] = jnp.where(rows < len_b, pe_ref[...], 0.0)

    col = jax.lax.broadcasted_iota(jnp.int32, (1, 1, MAX_SEQ_LEN), 2)
    pos_ref[...] = jnp.where(col < len_b, col + 1, 0)


def kernel(input_len, pe_table):
    pe = pe_table[1:]
    nj = MAX_SEQ_LEN // TJ
    emb, pos = pl.pallas_call(
        _emb_body,
        grid=(nj, BATCH),
        in_specs=[
            pl.BlockSpec(memory_space=pltpu.SMEM),
            pl.BlockSpec((TJ, D_MODEL), lambda j, b: (j, 0)),
        ],
        out_specs=[
            pl.BlockSpec((1, TJ, D_MODEL), lambda j, b: (b, j, 0)),
            pl.BlockSpec((1, 1, MAX_SEQ_LEN), lambda j, b: (b, 0, 0)),
        ],
        out_shape=[
            jax.ShapeDtypeStruct((BATCH, MAX_SEQ_LEN, D_MODEL), jnp.float32),
            jax.ShapeDtypeStruct((BATCH, 1, MAX_SEQ_LEN), jnp.int32),
        ],
    )(input_len, pe)
    return (emb, pos.reshape(BATCH, MAX_SEQ_LEN))


# speedup vs baseline: 4.6393x; 1.1814x over previous
import jax
import jax.numpy as jnp
from jax.experimental import pallas as pl
from jax.experimental.pallas import tpu as pltpu

D_MODEL = 1024
MAX_SEQ_LEN = 2048
BATCH = 16
TJ = 1024


def _emb_body(len_ref, pe_ref, emb_ref, pos_ref):
    j = pl.program_id(0)
    b = pl.program_id(1)
    len_b = len_ref[b]

    @pl.when(len_b >= (j + 1) * TJ)
    def _copy():
        emb_ref[0] = pe_ref[...]

    @pl.when(len_b <= j * TJ)
    def _zero():
        emb_ref[0] = jnp.zeros((TJ, D_MODEL), jnp.float32)

    @pl.when(jnp.logical_and(len_b > j * TJ, len_b < (j + 1) * TJ))
    def _straddle():
        rows = j * TJ + jax.lax.broadcasted_iota(jnp.int32, (TJ, 1), 0)
        emb_ref[0] = jnp.where(rows < len_b, pe_ref[...], 0.0)

    col = jax.lax.broadcasted_iota(jnp.int32, (1, 1, MAX_SEQ_LEN), 2)
    pos_ref[...] = jnp.where(col < len_b, col + 1, 0)


def kernel(input_len, pe_table):
    pe = pe_table[1:]
    nj = MAX_SEQ_LEN // TJ
    emb, pos = pl.pallas_call(
        _emb_body,
        grid=(nj, BATCH),
        in_specs=[
            pl.BlockSpec(memory_space=pltpu.SMEM),
            pl.BlockSpec((TJ, D_MODEL), lambda j, b: (j, 0)),
        ],
        out_specs=[
            pl.BlockSpec((1, TJ, D_MODEL), lambda j, b: (b, j, 0)),
            pl.BlockSpec((1, 1, MAX_SEQ_LEN), lambda j, b: (b, 0, 0)),
        ],
        out_shape=[
            jax.ShapeDtypeStruct((BATCH, MAX_SEQ_LEN, D_MODEL), jnp.float32),
            jax.ShapeDtypeStruct((BATCH, 1, MAX_SEQ_LEN), jnp.int32),
        ],
    )(input_len, pe)
    return (emb, pos.reshape(BATCH, MAX_SEQ_LEN))


# speedup vs baseline: 4.6846x; 1.0098x over previous
import jax
import jax.numpy as jnp
from jax.experimental import pallas as pl
from jax.experimental.pallas import tpu as pltpu

D_MODEL = 1024
MAX_SEQ_LEN = 2048
BATCH = 16
TJ = 2048


def _emb_body(len_ref, pe_ref, emb_ref, pos_ref):
    j = pl.program_id(0)
    b = pl.program_id(1)
    len_b = len_ref[b]

    @pl.when(len_b >= (j + 1) * TJ)
    def _copy():
        emb_ref[0] = pe_ref[...]

    @pl.when(len_b <= j * TJ)
    def _zero():
        emb_ref[0] = jnp.zeros((TJ, D_MODEL), jnp.float32)

    @pl.when(jnp.logical_and(len_b > j * TJ, len_b < (j + 1) * TJ))
    def _straddle():
        rows = j * TJ + jax.lax.broadcasted_iota(jnp.int32, (TJ, 1), 0)
        emb_ref[0] = jnp.where(rows < len_b, pe_ref[...], 0.0)

    col = jax.lax.broadcasted_iota(jnp.int32, (1, 1, MAX_SEQ_LEN), 2)
    pos_ref[...] = jnp.where(col < len_b, col + 1, 0)


def kernel(input_len, pe_table):
    pe = pe_table[1:]
    nj = MAX_SEQ_LEN // TJ
    emb, pos = pl.pallas_call(
        _emb_body,
        grid=(nj, BATCH),
        in_specs=[
            pl.BlockSpec(memory_space=pltpu.SMEM),
            pl.BlockSpec((TJ, D_MODEL), lambda j, b: (j, 0)),
        ],
        out_specs=[
            pl.BlockSpec((1, TJ, D_MODEL), lambda j, b: (b, j, 0)),
            pl.BlockSpec((1, 1, MAX_SEQ_LEN), lambda j, b: (b, 0, 0)),
        ],
        out_shape=[
            jax.ShapeDtypeStruct((BATCH, MAX_SEQ_LEN, D_MODEL), jnp.float32),
            jax.ShapeDtypeStruct((BATCH, 1, MAX_SEQ_LEN), jnp.int32),
        ],
    )(input_len, pe)
    return (emb, pos.reshape(BATCH, MAX_SEQ_LEN))
